# baseline (speedup 1.0000x reference)
import functools

import jax
import jax.numpy as jnp
from jax import lax
from jax.experimental import pallas as pl
from jax.experimental.pallas import tpu as pltpu
from jax.experimental.pallas import tpu_sc as plsc

N = 1048576
N_BINS = 20
LANES = 16
NC = 2
NS = 16
NW = NC * NS
CHUNK = N // NW
VECS = CHUNK // LANES
PART = 384

_mesh = plsc.VectorSubcoreMesh(core_axis_name="c", subcore_axis_name="s")
_params = pltpu.CompilerParams(needs_layout_passes=False)


@functools.partial(
    pl.kernel,
    out_type=jax.ShapeDtypeStruct((NC, PART), jnp.float32),
    mesh=_mesh,
    compiler_params=_params,
    scratch_types=[
        pltpu.VMEM((CHUNK,), jnp.float32),
        pltpu.VMEM((CHUNK,), jnp.float32),
        pltpu.VMEM((PART,), jnp.float32),
        pltpu.VMEM((NS, PART), jnp.float32),
        pltpu.VMEM((PART,), jnp.float32),
        pltpu.VMEM_SHARED((NS, PART), jnp.float32),
        pltpu.SemaphoreType.DMA,
        pltpu.SemaphoreType.DMA,
    ],
)
def _phase1(confs_hbm, accs_hbm, part_hbm, conf_v, acc_v, accum, red_v,
            total_v, shared, sem_c, sem_a):
    c_id = lax.axis_index("c")
    s_id = lax.axis_index("s")
    w = c_id * NS + s_id
    base = pl.multiple_of(w * CHUNK, CHUNK)
    pltpu.sync_copy(confs_hbm.at[pl.ds(base, CHUNK)], conf_v)
    pltpu.sync_copy(accs_hbm.at[pl.ds(base, CHUNK)], acc_v)

    for k in range(PART // LANES):
        accum[pl.ds(k * LANES, LANES)] = jnp.zeros((LANES,), jnp.float32)

    @plsc.parallel_loop(0, VECS, unroll=16)
    def body(i):
        lane = lax.iota(jnp.int32, LANES)
        off = pl.multiple_of(i * LANES, LANES)
        c = conf_v[pl.ds(off, LANES)]
        a = acc_v[pl.ds(off, LANES)]
        t = (c - 0.5) * 40.0
        b = jnp.clip(t.astype(jnp.int32), 0, N_BINS - 1)
        valid = c > 0.5
        idx = b * LANES + lane
        plsc.addupdate_scatter(accum, [idx], c - a, mask=valid)

    pltpu.sync_copy(accum, shared.at[s_id])
    plsc.subcore_barrier()

    @pl.when(s_id == 0)
    def _():
        pltpu.sync_copy(shared, red_v)
        for k in range(PART // LANES):
            sl = pl.ds(k * LANES, LANES)
            v = red_v[0, sl]
            for r in range(1, NS):
                v = v + red_v[r, sl]
            total_v[sl] = v
        pltpu.sync_copy(total_v, part_hbm.at[c_id])


def _epilogue_body(part_ref, out_ref):
    s = part_ref[0, :] + part_ref[1, :]
    ece = jnp.float32(0.0)
    for b in range(N_BINS):
        ece = ece + jnp.abs(
            jnp.sum(lax.slice(s, (b * LANES,), ((b + 1) * LANES,))))
    out_ref[0] = ece * jnp.float32(1.0 / N)


def _epilogue(part):
    return pl.pallas_call(
        _epilogue_body,
        out_shape=jax.ShapeDtypeStruct((1,), jnp.float32),
        out_specs=pl.BlockSpec(memory_space=pltpu.SMEM),
    )(part)


def kernel(confs, accs):
    part = _phase1(confs, accs)
    return _epilogue(part)


# speedup vs baseline: 1.0602x; 1.0602x over previous
import functools

import jax
import jax.numpy as jnp
from jax import lax
from jax.experimental import pallas as pl
from jax.experimental.pallas import tpu as pltpu
from jax.experimental.pallas import tpu_sc as plsc

N = 1048576
N_BINS = 20
LANES = 16
NC = 2
NS = 16
NW = NC * NS
CHUNK = N // NW
VECS = CHUNK // LANES
PART = 2 * N_BINS * LANES

_mesh = plsc.VectorSubcoreMesh(core_axis_name="c", subcore_axis_name="s")
_params = pltpu.CompilerParams(needs_layout_passes=False,
                               disable_bounds_checks=True)


@functools.partial(
    pl.kernel,
    out_type=jax.ShapeDtypeStruct((NC, PART), jnp.float32),
    mesh=_mesh,
    compiler_params=_params,
    scratch_types=[
        pltpu.VMEM((CHUNK,), jnp.float32),
        pltpu.VMEM((CHUNK,), jnp.float32),
        pltpu.VMEM((PART,), jnp.float32),
        pltpu.VMEM((NS, PART), jnp.float32),
        pltpu.VMEM((PART,), jnp.float32),
        pltpu.VMEM_SHARED((NS, PART), jnp.float32),
    ],
)
def _phase1(confs_hbm, accs_hbm, part_hbm, conf_v, acc_v, accum, red_v,
            total_v, shared):
    c_id = lax.axis_index("c")
    s_id = lax.axis_index("s")
    w = c_id * NS + s_id
    base = pl.multiple_of(w * CHUNK, CHUNK)
    pltpu.sync_copy(confs_hbm.at[pl.ds(base, CHUNK)], conf_v)
    pltpu.sync_copy(accs_hbm.at[pl.ds(base, CHUNK)], acc_v)

    for k in range(PART // LANES):
        accum[pl.ds(k * LANES, LANES)] = jnp.zeros((LANES,), jnp.float32)

    @plsc.parallel_loop(0, VECS, unroll=16)
    def body(i):
        lane = lax.iota(jnp.int32, LANES)
        off = pl.multiple_of(i * LANES, LANES)
        c = conf_v[pl.ds(off, LANES)]
        a = acc_v[pl.ds(off, LANES)]
        t = (c - 0.5) * 40.0
        b = jnp.clip(t.astype(jnp.int32), 0, N_BINS - 1)
        valid = c > 0.5
        idx = b * LANES + lane
        plsc.addupdate_scatter(accum, [idx], c, mask=valid)
        plsc.addupdate_scatter(accum, [idx + N_BINS * LANES], a, mask=valid)

    pltpu.sync_copy(accum, shared.at[s_id])
    plsc.subcore_barrier()

    @pl.when(s_id == 0)
    def _():
        pltpu.sync_copy(shared, red_v)
        for k in range(PART // LANES):
            sl = pl.ds(k * LANES, LANES)
            v = red_v[0, sl]
            for r in range(1, NS):
                v = v + red_v[r, sl]
            total_v[sl] = v
        pltpu.sync_copy(total_v, part_hbm.at[c_id])


def _epilogue_body(part_ref, out_ref):
    s = part_ref[0, :] + part_ref[1, :]
    ece = jnp.float32(0.0)
    for b in range(N_BINS):
        cv = jnp.sum(lax.slice(s, (b * LANES,), ((b + 1) * LANES,)))
        av = jnp.sum(lax.slice(s, ((N_BINS + b) * LANES,),
                               ((N_BINS + b + 1) * LANES,)))
        ece = ece + jnp.abs(cv - av)
    out_ref[0] = ece * jnp.float32(1.0 / N)


def _epilogue(part):
    return pl.pallas_call(
        _epilogue_body,
        out_shape=jax.ShapeDtypeStruct((1,), jnp.float32),
        out_specs=pl.BlockSpec(memory_space=pltpu.SMEM),
    )(part)


def kernel(confs, accs):
    part = _phase1(confs, accs)
    return _epilogue(part)


# speedup vs baseline: 1.1833x; 1.1162x over previous
import functools

import jax
import jax.numpy as jnp
from jax import lax
from jax.experimental import pallas as pl
from jax.experimental.pallas import tpu as pltpu
from jax.experimental.pallas import tpu_sc as plsc

N = 1048576
N_BINS = 20
LANES = 16
NC = 2
NS = 16
NW = NC * NS
CHUNK = N // NW
VECS = CHUNK // LANES
PART = 2 * N_BINS * LANES

_mesh = plsc.VectorSubcoreMesh(core_axis_name="c", subcore_axis_name="s")
_params = pltpu.CompilerParams(needs_layout_passes=False,
                               disable_bounds_checks=True)


@functools.partial(
    pl.kernel,
    out_type=jax.ShapeDtypeStruct((NW, PART), jnp.float32),
    mesh=_mesh,
    compiler_params=_params,
    scratch_types=[
        pltpu.VMEM((CHUNK,), jnp.float32),
        pltpu.VMEM((CHUNK,), jnp.float32),
        pltpu.VMEM((PART,), jnp.float32),
    ],
)
def _phase1(confs_hbm, accs_hbm, part_hbm, conf_v, acc_v, accum):
    c_id = lax.axis_index("c")
    s_id = lax.axis_index("s")
    w = c_id * NS + s_id
    base = pl.multiple_of(w * CHUNK, CHUNK)
    pltpu.sync_copy(confs_hbm.at[pl.ds(base, CHUNK)], conf_v)
    pltpu.sync_copy(accs_hbm.at[pl.ds(base, CHUNK)], acc_v)

    for k in range(PART // LANES):
        accum[pl.ds(k * LANES, LANES)] = jnp.zeros((LANES,), jnp.float32)

    @plsc.parallel_loop(0, VECS, unroll=16)
    def body(i):
        lane = lax.iota(jnp.int32, LANES)
        off = pl.multiple_of(i * LANES, LANES)
        c = conf_v[pl.ds(off, LANES)]
        a = acc_v[pl.ds(off, LANES)]
        t = (c - 0.5) * 40.0
        b = jnp.clip(t.astype(jnp.int32), 0, N_BINS - 1)
        valid = c > 0.5
        idx = b * LANES + lane
        plsc.addupdate_scatter(accum, [idx], c, mask=valid)
        plsc.addupdate_scatter(accum, [idx + N_BINS * LANES], a, mask=valid)

    pltpu.sync_copy(accum, part_hbm.at[w])


def _epilogue_body(part_ref, out_ref):
    s = jnp.sum(part_ref[...], axis=0)
    ece = jnp.float32(0.0)
    for b in range(N_BINS):
        cv = jnp.sum(lax.slice(s, (b * LANES,), ((b + 1) * LANES,)))
        av = jnp.sum(lax.slice(s, ((N_BINS + b) * LANES,),
                               ((N_BINS + b + 1) * LANES,)))
        ece = ece + jnp.abs(cv - av)
    out_ref[0] = ece * jnp.float32(1.0 / N)


def _epilogue(part):
    return pl.pallas_call(
        _epilogue_body,
        out_shape=jax.ShapeDtypeStruct((1,), jnp.float32),
        out_specs=pl.BlockSpec(memory_space=pltpu.SMEM),
    )(part)


def kernel(confs, accs):
    part = _phase1(confs, accs)
    return _epilogue(part)


# speedup vs baseline: 1.2098x; 1.0224x over previous
import functools

import jax
import jax.numpy as jnp
from jax import lax
from jax.experimental import pallas as pl
from jax.experimental.pallas import tpu as pltpu
from jax.experimental.pallas import tpu_sc as plsc

N = 1048576
N_BINS = 20
LANES = 16
NC = 2
NS = 16
NW = NC * NS
CHUNK = N // NW
VECS = CHUNK // LANES
NSUB = 4
SUB = CHUNK // NSUB
SVECS = SUB // LANES
PART = 2 * N_BINS * LANES

_mesh = plsc.VectorSubcoreMesh(core_axis_name="c", subcore_axis_name="s")
_params = pltpu.CompilerParams(needs_layout_passes=False,
                               disable_bounds_checks=True)


@functools.partial(
    pl.kernel,
    out_type=jax.ShapeDtypeStruct((NW, PART), jnp.float32),
    mesh=_mesh,
    compiler_params=_params,
    scratch_types=[
        pltpu.VMEM((CHUNK,), jnp.float32),
        pltpu.VMEM((CHUNK,), jnp.float32),
        pltpu.VMEM((PART,), jnp.float32),
        [pltpu.SemaphoreType.DMA] * 8,
    ],
)
def _phase1(confs_hbm, accs_hbm, part_hbm, conf_v, acc_v, accum, sems):
    c_id = lax.axis_index("c")
    s_id = lax.axis_index("s")
    w = c_id * NS + s_id
    base = pl.multiple_of(w * CHUNK, CHUNK)

    cps = {}
    for g in range(NSUB):
        sl_h = pl.ds(pl.multiple_of(base + g * SUB, SUB), SUB)
        sl_v = pl.ds(g * SUB, SUB)
        cps[g] = (
            pltpu.async_copy(confs_hbm.at[sl_h], conf_v.at[sl_v], sems[2 * g]),
            pltpu.async_copy(accs_hbm.at[sl_h], acc_v.at[sl_v], sems[2 * g + 1]),
        )
        if g == 0:
            for k in range(PART // LANES):
                accum[pl.ds(k * LANES, LANES)] = jnp.zeros((LANES,),
                                                           jnp.float32)

    for g in range(NSUB):
        cps[g][0].wait()
        cps[g][1].wait()

        @plsc.parallel_loop(g * SVECS, (g + 1) * SVECS, unroll=16)
        def body(i):
            lane = lax.iota(jnp.int32, LANES)
            off = pl.multiple_of(i * LANES, LANES)
            c = conf_v[pl.ds(off, LANES)]
            a = acc_v[pl.ds(off, LANES)]
            t = (c - 0.5) * 40.0
            b = jnp.clip(t.astype(jnp.int32), 0, N_BINS - 1)
            valid = c > 0.5
            idx = b * LANES + lane
            plsc.addupdate_scatter(accum, [idx], c, mask=valid)
            plsc.addupdate_scatter(accum, [idx + N_BINS * LANES], a,
                                   mask=valid)

    pltpu.sync_copy(accum, part_hbm.at[w])


def _epilogue_body(part_ref, out_ref):
    s = jnp.sum(part_ref[...], axis=0)
    ece = jnp.float32(0.0)
    for b in range(N_BINS):
        cv = jnp.sum(lax.slice(s, (b * LANES,), ((b + 1) * LANES,)))
        av = jnp.sum(lax.slice(s, ((N_BINS + b) * LANES,),
                               ((N_BINS + b + 1) * LANES,)))
        ece = ece + jnp.abs(cv - av)
    out_ref[0] = ece * jnp.float32(1.0 / N)


def _epilogue(part):
    return pl.pallas_call(
        _epilogue_body,
        out_shape=jax.ShapeDtypeStruct((1,), jnp.float32),
        out_specs=pl.BlockSpec(memory_space=pltpu.SMEM),
    )(part)


def kernel(confs, accs):
    part = _phase1(confs, accs)
    return _epilogue(part)
